# Initial kernel scaffold; baseline (speedup 1.0000x reference)
import functools

import jax
import jax.numpy as jnp
from jax import lax
from jax.experimental import pallas as pl
from jax.experimental.pallas import tpu as pltpu
from jax.experimental.pallas import tpu_sc as plsc

G1 = 34
PLANE = G1 * G1
NPLANES = 4 * G1
V = NPLANES * PLANE
V_PAD = 157696
VT = V_PAD // 16
SUBZ = 2464
DUMMY = 157248

C_AUG = 264
NCHUNK = 33

N_PAD = 102400
PT = N_PAD // 16
SR = 5
PR = 1280
PW = N_PAD // 32

_mesh = plsc.VectorSubcoreMesh(core_axis_name="c", subcore_axis_name="s")


def _k1_body(f_ref, x_ref, wpre_ref, g_ref, b_ref, wpos_ref, o_ref):
    h = jnp.dot(f_ref[...], wpre_ref[...], preferred_element_type=jnp.float32)
    mu = jnp.mean(h, axis=1, keepdims=True)
    xc = h - mu
    var = jnp.mean(xc * xc, axis=1, keepdims=True)
    F = xc * lax.rsqrt(var + 1e-6) * g_ref[...] + b_ref[...]
    pos = jnp.dot(x_ref[...], wpos_ref[...], preferred_element_type=jnp.float32)
    cw = jnp.cos(pos)
    sw = jnp.sin(pos)
    br = F.shape[0]
    ones = jnp.ones((br, 1), jnp.float32)
    zer = jnp.zeros((br, C_AUG - 257), jnp.float32)
    o_ref[...] = jnp.concatenate([F * cw, F * sw, ones, zer], axis=1)


def _k1(feats_p, xyz8, wpreT, gamma, beta, wpos8):
    br = 1024
    return pl.pallas_call(
        _k1_body,
        grid=(N_PAD // br,),
        in_specs=[
            pl.BlockSpec((br, 128), lambda i: (i, 0)),
            pl.BlockSpec((br, 8), lambda i: (i, 0)),
            pl.BlockSpec((128, 128), lambda i: (0, 0)),
            pl.BlockSpec((1, 128), lambda i: (0, 0)),
            pl.BlockSpec((1, 128), lambda i: (0, 0)),
            pl.BlockSpec((8, 128), lambda i: (0, 0)),
        ],
        out_specs=pl.BlockSpec((br, C_AUG), lambda i: (i, 0)),
        out_shape=jax.ShapeDtypeStruct((N_PAD, C_AUG), jnp.float32),
    )(feats_p, xyz8, wpreT, gamma, beta, wpos8)


def _shift_rows(a, k):
    if k > 0:
        return jnp.concatenate(
            [a[k:], jnp.zeros((k, a.shape[1]), a.dtype)], axis=0)
    return jnp.concatenate(
        [jnp.zeros((-k, a.shape[1]), a.dtype), a[:k]], axis=0)


def _k3_body(g_ref, o_ref, scr):
    x = g_ref[...]
    s = x + _shift_rows(x, 1) + _shift_rows(x, -1)
    s = s + _shift_rows(s, G1) + _shift_rows(s, -G1)
    s0 = s[:PLANE]
    s1 = s[PLANE:]
    out0 = scr[2] + scr[0] + scr[1]
    out1 = scr[0] + scr[1] + s0
    acc = jnp.concatenate([out0, out1], axis=0)
    o_ref[...] = acc[:, :256] / acc[:, 256:257]
    scr[2] = scr[1]
    scr[0] = s0
    scr[1] = s1


def _k3(grid_arr):
    nblk = NPLANES // 2
    return pl.pallas_call(
        _k3_body,
        grid=(nblk + 1,),
        in_specs=[pl.BlockSpec((2 * PLANE, C_AUG),
                               lambda i: (jnp.minimum(i, nblk - 1), 0))],
        out_specs=pl.BlockSpec((2 * PLANE, 256),
                               lambda i: (jnp.maximum(i - 1, 0), 0)),
        out_shape=jax.ShapeDtypeStruct((V_PAD, 256), jnp.float32),
        scratch_shapes=[pltpu.VMEM((3, PLANE, C_AUG), jnp.float32)],
    )(grid_arr)


def _k5_body(g_ref, x_ref, wpos_ref, o_ref):
    pos = jnp.dot(x_ref[...], wpos_ref[...], preferred_element_type=jnp.float32)
    g = g_ref[...]
    o_ref[...] = g[:, :128] * jnp.cos(pos) + g[:, 128:256] * jnp.sin(pos)


def _k5(gathered, xyz8, wpos8):
    br = 1024
    return pl.pallas_call(
        _k5_body,
        grid=(N_PAD // br,),
        in_specs=[
            pl.BlockSpec((br, 256), lambda i: (i, 0)),
            pl.BlockSpec((br, 8), lambda i: (i, 0)),
            pl.BlockSpec((8, 128), lambda i: (0, 0)),
        ],
        out_specs=pl.BlockSpec((br, 128), lambda i: (i, 0)),
        out_shape=jax.ShapeDtypeStruct((N_PAD, 128), jnp.float32),
    )(gathered, xyz8, wpos8)


def _k2_body(fw_hbm, idx_hbm, zz_hbm, grid_hbm, spg, zbuf, ibuf, vbuf, obuf):
    c = lax.axis_index("c")
    s = lax.axis_index("s")
    pltpu.sync_copy(zz_hbm, zbuf)
    row0 = s * VT

    def chunk_iter(jj, carry):
        chunk = jj * 2 + c
        valid = chunk < NCHUNK
        col0 = chunk * 8

        @pl.when(valid)
        def _zero():
            def zr(r, cc):
                pltpu.sync_copy(zbuf, spg.at[pl.ds(row0 + r * SUBZ, SUBZ)])
                return cc
            lax.fori_loop(0, 4, zr, 0)

        plsc.subcore_barrier()

        @pl.when(valid)
        def _scatter():
            def sr(r, cc):
                p0 = s * PT + r * PR
                pltpu.sync_copy(idx_hbm.at[pl.ds((s * PT) // 128 + r * 10, 10)],
                                ibuf)
                pltpu.sync_copy(fw_hbm.at[pl.ds(p0, PR), pl.ds(col0, 8)], vbuf)
                for q in range(10):
                    pltpu.sync_copy(vbuf.at[pl.ds(q * 128, 128)],
                                    spg.at[ibuf.at[q]], add=True)
                return cc
            lax.fori_loop(0, SR, sr, 0)

        plsc.subcore_barrier()

        @pl.when(valid)
        def _readout():
            def ro(r, cc):
                rr = row0 + r * SUBZ
                pltpu.sync_copy(spg.at[pl.ds(rr, SUBZ)], obuf)
                pltpu.sync_copy(obuf,
                                grid_hbm.at[pl.ds(rr, SUBZ), pl.ds(col0, 8)])
                return cc
            lax.fori_loop(0, 4, ro, 0)

        plsc.subcore_barrier()
        return carry

    lax.fori_loop(0, 17, chunk_iter, 0)


_k2 = functools.partial(
    pl.kernel,
    out_type=jax.ShapeDtypeStruct((V_PAD, C_AUG), jnp.float32),
    mesh=_mesh,
    scratch_types=[
        pltpu.VMEM_SHARED((V_PAD, 8), jnp.float32),
        pltpu.VMEM((SUBZ, 8), jnp.float32),
        pltpu.VMEM((10, 128), jnp.int32),
        pltpu.VMEM((PR, 8), jnp.float32),
        pltpu.VMEM((SUBZ, 8), jnp.float32),
    ],
)(_k2_body)


def _k4_body(anorm_hbm, idx_hbm, out_hbm, ibuf, rbuf, sem):
    c = lax.axis_index("c")
    s = lax.axis_index("s")
    wid = s * 2 + c
    pltpu.sync_copy(idx_hbm.at[pl.ds(wid * 25, 25)], ibuf)

    def it(j, carry):
        pltpu.async_copy(anorm_hbm.at[ibuf.at[j]], rbuf, sem).wait()
        pltpu.sync_copy(rbuf, out_hbm.at[pl.ds(wid * PW + j * 128, 128)])
        return carry

    lax.fori_loop(0, 25, it, 0)


_k4 = functools.partial(
    pl.kernel,
    out_type=jax.ShapeDtypeStruct((N_PAD, 256), jnp.float32),
    mesh=_mesh,
    scratch_types=[
        pltpu.VMEM((25, 128), jnp.int32),
        pltpu.VMEM((128, 256), jnp.float32),
        pltpu.SemaphoreType.DMA,
    ],
)(_k4_body)


def kernel(feats, coords, W_pre, ln_gamma, ln_beta, W_pos, stride):
    n = feats.shape[0]
    xyz = coords[:, :3]
    bcol = coords[:, 3]
    small = xyz // stride
    ids = (((bcol * G1 + small[:, 0] + 1) * G1 + small[:, 1] + 1) * G1
           + small[:, 2] + 1).astype(jnp.int32)
    ids_p = jnp.concatenate(
        [ids, jnp.full((N_PAD - n,), DUMMY, jnp.int32)])
    idx2 = ids_p.reshape(N_PAD // 128, 128)

    feats_p = jnp.pad(feats, ((0, N_PAD - n), (0, 0)))
    xyz8 = jnp.pad(xyz.astype(jnp.float32), ((0, N_PAD - n), (0, 5)))
    wpreT = W_pre.T
    wpos8 = jnp.pad(W_pos.T.astype(jnp.float32), ((0, 5), (0, 0)))
    gamma = ln_gamma.reshape(1, 128)
    beta = ln_beta.reshape(1, 128)
    zz = jnp.zeros((SUBZ, 8), jnp.float32)

    fw_aug = _k1(feats_p, xyz8, wpreT, gamma, beta, wpos8)
    grid_arr = _k2(fw_aug, idx2, zz)
    anorm = _k3(grid_arr)
    gathered = _k4(anorm, idx2)
    out_p = _k5(gathered, xyz8, wpos8)
    return out_p[:n]


# speedup vs baseline: 52.9760x; 52.9760x over previous
import functools

import jax
import jax.numpy as jnp
from jax import lax
from jax.experimental import pallas as pl
from jax.experimental.pallas import tpu as pltpu
from jax.experimental.pallas import tpu_sc as plsc

G1 = 34
PLANE = G1 * G1
NPLANES = 4 * G1
V = NPLANES * PLANE
V_PAD = 157696
VT = V_PAD // 16
SUBZ = 2464
DUMMY = 157248

C_AUG = 264
NCHUNK = 33

N_PAD = 102400
PT = N_PAD // 16
SR = 5
PR = 1280
PW = N_PAD // 32


@functools.cache
def _mesh():
    return plsc.VectorSubcoreMesh(core_axis_name="c", subcore_axis_name="s")


def _k1_body(f_ref, x_ref, wpre_ref, g_ref, b_ref, wpos_ref, o_ref):
    h = jnp.dot(f_ref[...], wpre_ref[...], preferred_element_type=jnp.float32)
    mu = jnp.mean(h, axis=1, keepdims=True)
    xc = h - mu
    var = jnp.mean(xc * xc, axis=1, keepdims=True)
    F = xc * lax.rsqrt(var + 1e-6) * g_ref[...] + b_ref[...]
    pos = jnp.dot(x_ref[...], wpos_ref[...], preferred_element_type=jnp.float32)
    cw = jnp.cos(pos)
    sw = jnp.sin(pos)
    br = F.shape[0]
    ones = jnp.ones((br, 1), jnp.float32)
    zer = jnp.zeros((br, C_AUG - 257), jnp.float32)
    o_ref[...] = jnp.concatenate([F * cw, F * sw, ones, zer], axis=1)


def _k1(feats_p, xyz8, wpreT, gamma, beta, wpos8):
    br = 1024
    return pl.pallas_call(
        _k1_body,
        grid=(N_PAD // br,),
        in_specs=[
            pl.BlockSpec((br, 128), lambda i: (i, 0)),
            pl.BlockSpec((br, 8), lambda i: (i, 0)),
            pl.BlockSpec((128, 128), lambda i: (0, 0)),
            pl.BlockSpec((1, 128), lambda i: (0, 0)),
            pl.BlockSpec((1, 128), lambda i: (0, 0)),
            pl.BlockSpec((8, 128), lambda i: (0, 0)),
        ],
        out_specs=pl.BlockSpec((br, C_AUG), lambda i: (i, 0)),
        out_shape=jax.ShapeDtypeStruct((N_PAD, C_AUG), jnp.float32),
    )(feats_p, xyz8, wpreT, gamma, beta, wpos8)


def _shift_rows(a, k):
    if k > 0:
        return jnp.concatenate(
            [a[k:], jnp.zeros((k, a.shape[1]), a.dtype)], axis=0)
    return jnp.concatenate(
        [jnp.zeros((-k, a.shape[1]), a.dtype), a[:k]], axis=0)


def _k3_body(g_ref, o_ref, scr):
    x = g_ref[...]
    s = x + _shift_rows(x, 1) + _shift_rows(x, -1)
    s = s + _shift_rows(s, G1) + _shift_rows(s, -G1)
    s0 = s[:PLANE]
    s1 = s[PLANE:]
    out0 = scr[2] + scr[0] + scr[1]
    out1 = scr[0] + scr[1] + s0
    acc = jnp.concatenate([out0, out1], axis=0)
    o_ref[...] = acc[:, :256] / acc[:, 256:257]
    scr[2] = scr[1]
    scr[0] = s0
    scr[1] = s1


def _k3(grid_arr):
    nblk = NPLANES // 2
    return pl.pallas_call(
        _k3_body,
        grid=(nblk + 1,),
        in_specs=[pl.BlockSpec((2 * PLANE, C_AUG),
                               lambda i: (jnp.minimum(i, nblk - 1), 0))],
        out_specs=pl.BlockSpec((2 * PLANE, 256),
                               lambda i: (jnp.maximum(i - 1, 0), 0)),
        out_shape=jax.ShapeDtypeStruct((V_PAD, 256), jnp.float32),
        scratch_shapes=[pltpu.VMEM((3, PLANE, C_AUG), jnp.float32)],
    )(grid_arr)


def _k5_body(g_ref, x_ref, wpos_ref, o_ref):
    pos = jnp.dot(x_ref[...], wpos_ref[...], preferred_element_type=jnp.float32)
    g = g_ref[...]
    o_ref[...] = g[:, :128] * jnp.cos(pos) + g[:, 128:256] * jnp.sin(pos)


def _k5(gathered, xyz8, wpos8):
    br = 1024
    return pl.pallas_call(
        _k5_body,
        grid=(N_PAD // br,),
        in_specs=[
            pl.BlockSpec((br, 256), lambda i: (i, 0)),
            pl.BlockSpec((br, 8), lambda i: (i, 0)),
            pl.BlockSpec((8, 128), lambda i: (0, 0)),
        ],
        out_specs=pl.BlockSpec((br, 128), lambda i: (i, 0)),
        out_shape=jax.ShapeDtypeStruct((N_PAD, 128), jnp.float32),
    )(gathered, xyz8, wpos8)


def _k2_body(fw_hbm, idx_hbm, zz_hbm, grid_hbm, spg, zbuf, ibuf, vbuf, obuf):
    c = lax.axis_index("c")
    s = lax.axis_index("s")
    pltpu.sync_copy(zz_hbm, zbuf)
    row0 = s * VT

    def chunk_iter(jj, carry):
        chunk = jj * 2 + c
        valid = chunk < NCHUNK
        col0 = chunk * 8

        @pl.when(valid)
        def _zero():
            def zr(r, cc):
                pltpu.sync_copy(zbuf, spg.at[pl.ds(row0 + r * SUBZ, SUBZ)])
                return cc
            lax.fori_loop(0, 4, zr, 0)

        plsc.subcore_barrier()

        @pl.when(valid)
        def _scatter():
            def sr(r, cc):
                p0 = s * PT + r * PR
                pltpu.sync_copy(idx_hbm.at[pl.ds((s * PT) // 128 + r * 10, 10)],
                                ibuf)
                pltpu.sync_copy(fw_hbm.at[pl.ds(p0, PR), pl.ds(col0, 8)], vbuf)
                for q in range(10):
                    pltpu.sync_copy(vbuf.at[pl.ds(q * 128, 128)],
                                    spg.at[ibuf.at[q]], add=True)
                return cc
            lax.fori_loop(0, SR, sr, 0)

        plsc.subcore_barrier()

        @pl.when(valid)
        def _readout():
            def ro(r, cc):
                rr = row0 + r * SUBZ
                pltpu.sync_copy(spg.at[pl.ds(rr, SUBZ)], obuf)
                pltpu.sync_copy(obuf,
                                grid_hbm.at[pl.ds(rr, SUBZ), pl.ds(col0, 8)])
                return cc
            lax.fori_loop(0, 4, ro, 0)

        plsc.subcore_barrier()
        return carry

    lax.fori_loop(0, 17, chunk_iter, 0)


@functools.cache
def _k2():
    return pl.kernel(
        _k2_body,
        out_type=jax.ShapeDtypeStruct((V_PAD, C_AUG), jnp.float32),
        mesh=_mesh(),
        compiler_params=pltpu.CompilerParams(use_tc_tiling_on_sc=False),
        scratch_types=[
            pltpu.VMEM_SHARED((V_PAD, 8), jnp.float32),
            pltpu.VMEM((SUBZ, 8), jnp.float32),
            pltpu.VMEM((10, 128), jnp.int32),
            pltpu.VMEM((PR, 8), jnp.float32),
            pltpu.VMEM((SUBZ, 8), jnp.float32),
        ],
    )


def _k4_body(anorm_hbm, idx_hbm, out_hbm, ibuf, rbuf, sem):
    c = lax.axis_index("c")
    s = lax.axis_index("s")
    wid = s * 2 + c
    pltpu.sync_copy(idx_hbm.at[pl.ds(wid * 25, 25)], ibuf)

    def it(j, carry):
        pltpu.async_copy(anorm_hbm.at[ibuf.at[j]], rbuf, sem).wait()
        pltpu.sync_copy(rbuf, out_hbm.at[pl.ds(wid * PW + j * 128, 128)])
        return carry

    lax.fori_loop(0, 25, it, 0)


@functools.cache
def _k4():
    return pl.kernel(
        _k4_body,
        out_type=jax.ShapeDtypeStruct((N_PAD, 256), jnp.float32),
        mesh=_mesh(),
        compiler_params=pltpu.CompilerParams(use_tc_tiling_on_sc=False),
        scratch_types=[
            pltpu.VMEM((25, 128), jnp.int32),
            pltpu.VMEM((128, 256), jnp.float32),
            pltpu.SemaphoreType.DMA,
        ],
    )


def kernel(feats, coords, W_pre, ln_gamma, ln_beta, W_pos, stride):
    n = feats.shape[0]
    xyz = coords[:, :3]
    bcol = coords[:, 3]
    small = xyz // stride
    ids = (((bcol * G1 + small[:, 0] + 1) * G1 + small[:, 1] + 1) * G1
           + small[:, 2] + 1).astype(jnp.int32)
    ids_p = jnp.concatenate(
        [ids, jnp.full((N_PAD - n,), DUMMY, jnp.int32)])
    idx2 = ids_p.reshape(N_PAD // 128, 128)

    feats_p = jnp.pad(feats, ((0, N_PAD - n), (0, 0)))
    xyz8 = jnp.pad(xyz.astype(jnp.float32), ((0, N_PAD - n), (0, 5)))
    wpreT = W_pre.T
    wpos8 = jnp.pad(W_pos.T.astype(jnp.float32), ((0, 5), (0, 0)))
    gamma = ln_gamma.reshape(1, 128)
    beta = ln_beta.reshape(1, 128)
    zz = jnp.zeros((SUBZ, 8), jnp.float32)

    fw_aug = _k1(feats_p, xyz8, wpreT, gamma, beta, wpos8)
    grid_arr = _k2()(fw_aug, idx2, zz)
    anorm = _k3(grid_arr)
    gathered = _k4()(anorm, idx2)
    out_p = _k5(gathered, xyz8, wpos8)
    return out_p[:n]


# speedup vs baseline: 59.1726x; 1.1170x over previous
import functools

import jax
import jax.numpy as jnp
from jax import lax
from jax.experimental import pallas as pl
from jax.experimental.pallas import tpu as pltpu
from jax.experimental.pallas import tpu_sc as plsc

G1 = 34
PLANE = G1 * G1
NPLANES = 4 * G1
V = NPLANES * PLANE
V_PAD = 157696
VT = V_PAD // 16
SUBZ = 1232
DUMMY = 157248

C_AUG = 264
NCHUNK = 33

N_PAD = 102400
PT = N_PAD // 16
SR = 5
PR = 1280
PW = N_PAD // 32


@functools.cache
def _mesh():
    return plsc.VectorSubcoreMesh(core_axis_name="c", subcore_axis_name="s")


_PI_HI = 3.1415927410125732
_PI_LO = -8.742278012618954e-08
_S1, _S3, _S5, _S7, _S9 = (0.999999997, -0.1666666, 8.33309755e-3,
                           -1.98124848e-4, 2.61290035e-6)
_C0, _C2, _C4, _C6, _C8, _C10 = (1.0, -0.499999995, 4.16666419e-2,
                                 -1.38884323e-3, 2.47637666e-5,
                                 -2.61149497e-7)


def _sincos(x):
    n = jnp.floor(x * (1.0 / jnp.pi) + 0.5)
    r = (x - n * _PI_HI) - n * _PI_LO
    sign = 1.0 - 2.0 * (n - 2.0 * jnp.floor(0.5 * n))
    s = r * r
    sp = ((((_S9 * s + _S7) * s + _S5) * s + _S3) * s + _S1) * r
    cp = ((((_C10 * s + _C8) * s + _C6) * s + _C4) * s + _C2) * s + _C0
    return sign * sp, sign * cp


def _k1_body(f_ref, x_ref, wpre_ref, g_ref, b_ref, wpos_ref, o_ref):
    h = jnp.dot(f_ref[...], wpre_ref[...], preferred_element_type=jnp.float32)
    mu = jnp.mean(h, axis=1, keepdims=True)
    xc = h - mu
    var = jnp.mean(xc * xc, axis=1, keepdims=True)
    F = xc * lax.rsqrt(var + 1e-6) * g_ref[...] + b_ref[...]
    pos = jnp.dot(x_ref[...], wpos_ref[...], preferred_element_type=jnp.float32)
    sw, cw = _sincos(pos)
    br = F.shape[0]
    ones = jnp.ones((br, 1), jnp.float32)
    zer = jnp.zeros((br, C_AUG - 257), jnp.float32)
    o_ref[...] = jnp.concatenate([F * cw, F * sw, ones, zer], axis=1)


def _k1(feats_p, xyz8, wpreT, gamma, beta, wpos8):
    br = 1024
    return pl.pallas_call(
        _k1_body,
        grid=(N_PAD // br,),
        in_specs=[
            pl.BlockSpec((br, 128), lambda i: (i, 0)),
            pl.BlockSpec((br, 8), lambda i: (i, 0)),
            pl.BlockSpec((128, 128), lambda i: (0, 0)),
            pl.BlockSpec((1, 128), lambda i: (0, 0)),
            pl.BlockSpec((1, 128), lambda i: (0, 0)),
            pl.BlockSpec((8, 128), lambda i: (0, 0)),
        ],
        out_specs=pl.BlockSpec((br, C_AUG), lambda i: (i, 0)),
        out_shape=jax.ShapeDtypeStruct((N_PAD, C_AUG), jnp.float32),
    )(feats_p, xyz8, wpreT, gamma, beta, wpos8)


def _shift_rows(a, k):
    if k > 0:
        return jnp.concatenate(
            [a[k:], jnp.zeros((k, a.shape[1]), a.dtype)], axis=0)
    return jnp.concatenate(
        [jnp.zeros((-k, a.shape[1]), a.dtype), a[:k]], axis=0)


def _k3_body(g_ref, o_ref, scr):
    x = g_ref[...]
    s = x + _shift_rows(x, 1) + _shift_rows(x, -1)
    s = s + _shift_rows(s, G1) + _shift_rows(s, -G1)
    s0 = s[:PLANE]
    s1 = s[PLANE:]
    out0 = scr[2] + scr[0] + scr[1]
    out1 = scr[0] + scr[1] + s0
    acc = jnp.concatenate([out0, out1], axis=0)
    o_ref[...] = acc[:, :256] / acc[:, 256:257]
    scr[2] = scr[1]
    scr[0] = s0
    scr[1] = s1


def _k3(grid_arr):
    nblk = NPLANES // 2
    return pl.pallas_call(
        _k3_body,
        grid=(nblk + 1,),
        in_specs=[pl.BlockSpec((2 * PLANE, C_AUG),
                               lambda i: (jnp.minimum(i, nblk - 1), 0))],
        out_specs=pl.BlockSpec((2 * PLANE, 256),
                               lambda i: (jnp.maximum(i - 1, 0), 0)),
        out_shape=jax.ShapeDtypeStruct((V_PAD, 256), jnp.float32),
        scratch_shapes=[pltpu.VMEM((3, PLANE, C_AUG), jnp.float32)],
    )(grid_arr)


def _k5_body(g_ref, x_ref, wpos_ref, o_ref):
    pos = jnp.dot(x_ref[...], wpos_ref[...], preferred_element_type=jnp.float32)
    sw, cw = _sincos(pos)
    g = g_ref[...]
    o_ref[...] = g[:, :128] * cw + g[:, 128:256] * sw


def _k5(gathered, xyz8, wpos8):
    br = 1024
    return pl.pallas_call(
        _k5_body,
        grid=(N_PAD // br,),
        in_specs=[
            pl.BlockSpec((br, 256), lambda i: (i, 0)),
            pl.BlockSpec((br, 8), lambda i: (i, 0)),
            pl.BlockSpec((8, 128), lambda i: (0, 0)),
        ],
        out_specs=pl.BlockSpec((br, 128), lambda i: (i, 0)),
        out_shape=jax.ShapeDtypeStruct((N_PAD, 128), jnp.float32),
    )(gathered, xyz8, wpos8)


def _k2_body(fw_hbm, idx_hbm, zz_hbm, grid_hbm, spg, zbuf, ibuf, vbufa, vbufb,
             obuf, sema, semb):
    c = lax.axis_index("c")
    s = lax.axis_index("s")
    pltpu.sync_copy(zz_hbm, zbuf)
    pltpu.sync_copy(idx_hbm.at[pl.ds(s * 50, 50)], ibuf)
    row0 = s * VT
    p_base = s * PT

    def chunk_iter(jj, carry):
        chunk = jj * 2 + c
        valid = chunk < NCHUNK
        col0 = chunk * 8

        def _src(r):
            return fw_hbm.at[pl.ds(p_base + r * PR, PR), pl.ds(col0, 8)]

        def _start(r, buf, sem):
            pltpu.async_copy(_src(r), buf, sem)

        def _wait(r, buf, sem):
            pltpu.make_async_copy(_src(r), buf, sem).wait()

        def _scatter10(r, buf):
            for q in range(10):
                pltpu.sync_copy(buf.at[pl.ds(q * 128, 128)],
                                spg.at[ibuf.at[r * 10 + q]], add=True)

        @pl.when(valid)
        def _zero():
            def zr(r, cc):
                pltpu.sync_copy(zbuf, spg.at[pl.ds(row0 + r * SUBZ, SUBZ)])
                return cc
            lax.fori_loop(0, 8, zr, 0)

        plsc.subcore_barrier()

        @pl.when(valid)
        def _scatter():
            _start(0, vbufa, sema)

            def pair(g, cc):
                r = g * 2
                _start(r + 1, vbufb, semb)
                _wait(r, vbufa, sema)
                _scatter10(r, vbufa)
                _start(r + 2, vbufa, sema)
                _wait(r + 1, vbufb, semb)
                _scatter10(r + 1, vbufb)
                return cc
            lax.fori_loop(0, 2, pair, 0)
            _wait(4, vbufa, sema)
            _scatter10(4, vbufa)

        plsc.subcore_barrier()

        @pl.when(valid)
        def _readout():
            def ro(r, cc):
                rr = row0 + r * SUBZ
                pltpu.sync_copy(spg.at[pl.ds(rr, SUBZ)], obuf)
                pltpu.sync_copy(obuf,
                                grid_hbm.at[pl.ds(rr, SUBZ), pl.ds(col0, 8)])
                return cc
            lax.fori_loop(0, 8, ro, 0)

        plsc.subcore_barrier()
        return carry

    lax.fori_loop(0, 17, chunk_iter, 0)


@functools.cache
def _k2():
    return pl.kernel(
        _k2_body,
        out_type=jax.ShapeDtypeStruct((V_PAD, C_AUG), jnp.float32),
        mesh=_mesh(),
        compiler_params=pltpu.CompilerParams(use_tc_tiling_on_sc=False),
        scratch_types=[
            pltpu.VMEM_SHARED((V_PAD, 8), jnp.float32),
            pltpu.VMEM((SUBZ, 8), jnp.float32),
            pltpu.VMEM((50, 128), jnp.int32),
            pltpu.VMEM((PR, 8), jnp.float32),
            pltpu.VMEM((PR, 8), jnp.float32),
            pltpu.VMEM((SUBZ, 8), jnp.float32),
            pltpu.SemaphoreType.DMA,
            pltpu.SemaphoreType.DMA,
        ],
    )


def _k4_body(anorm_hbm, idx_hbm, out_hbm, ibuf, rbufa, rbufb, sema, semb):
    c = lax.axis_index("c")
    s = lax.axis_index("s")
    wid = s * 2 + c
    base = wid * PW
    pltpu.sync_copy(idx_hbm.at[pl.ds(wid * 25, 25)], ibuf)

    def _start(k, buf, sem):
        pltpu.async_copy(anorm_hbm.at[ibuf.at[k]], buf, sem)

    def _wait(k, buf, sem):
        pltpu.make_async_copy(anorm_hbm.at[ibuf.at[k]], buf, sem).wait()

    def _flush(k, buf):
        pltpu.sync_copy(buf, out_hbm.at[pl.ds(base + k * 128, 128)])

    _start(0, rbufa, sema)

    def pair(g, carry):
        j = g * 2
        _start(j + 1, rbufb, semb)
        _wait(j, rbufa, sema)
        _flush(j, rbufa)
        _start(j + 2, rbufa, sema)
        _wait(j + 1, rbufb, semb)
        _flush(j + 1, rbufb)
        return carry

    lax.fori_loop(0, 12, pair, 0)
    _wait(24, rbufa, sema)
    _flush(24, rbufa)


@functools.cache
def _k4():
    return pl.kernel(
        _k4_body,
        out_type=jax.ShapeDtypeStruct((N_PAD, 256), jnp.float32),
        mesh=_mesh(),
        compiler_params=pltpu.CompilerParams(use_tc_tiling_on_sc=False),
        scratch_types=[
            pltpu.VMEM((25, 128), jnp.int32),
            pltpu.VMEM((128, 256), jnp.float32),
            pltpu.VMEM((128, 256), jnp.float32),
            pltpu.SemaphoreType.DMA,
            pltpu.SemaphoreType.DMA,
        ],
    )


def kernel(feats, coords, W_pre, ln_gamma, ln_beta, W_pos, stride):
    n = feats.shape[0]
    xyz = coords[:, :3]
    bcol = coords[:, 3]
    small = xyz // stride
    ids = (((bcol * G1 + small[:, 0] + 1) * G1 + small[:, 1] + 1) * G1
           + small[:, 2] + 1).astype(jnp.int32)
    ids_p = jnp.concatenate(
        [ids, jnp.full((N_PAD - n,), DUMMY, jnp.int32)])
    idx2 = ids_p.reshape(N_PAD // 128, 128)

    feats_p = jnp.pad(feats, ((0, N_PAD - n), (0, 0)))
    xyz8 = jnp.pad(xyz.astype(jnp.float32), ((0, N_PAD - n), (0, 5)))
    wpreT = W_pre.T
    wpos8 = jnp.pad(W_pos.T.astype(jnp.float32), ((0, 5), (0, 0)))
    gamma = ln_gamma.reshape(1, 128)
    beta = ln_beta.reshape(1, 128)
    zz = jnp.zeros((SUBZ, 8), jnp.float32)

    fw_aug = _k1(feats_p, xyz8, wpreT, gamma, beta, wpos8)
    grid_arr = _k2()(fw_aug, idx2, zz)
    anorm = _k3(grid_arr)
    gathered = _k4()(anorm, idx2)
    out_p = _k5(gathered, xyz8, wpos8)
    return out_p[:n]


# speedup vs baseline: 107.9139x; 1.8237x over previous
import functools

import jax
import jax.numpy as jnp
from jax import lax
from jax.experimental import pallas as pl
from jax.experimental.pallas import tpu as pltpu
from jax.experimental.pallas import tpu_sc as plsc

G1 = 34
PLANE = G1 * G1
NPLANES = 4 * G1
V = NPLANES * PLANE
V_PAD = 158464
VT = V_PAD // 16
SUBZ = 1238
DUMMY = 158407

C_AUG = 264
NCHUNK = 33

N_PROC = 100352
BR = 1024
NBLK = N_PROC // BR
PT = N_PROC // 16
SR = 7
PR = 896
NW4 = 28
PW = N_PROC // NW4


@functools.cache
def _mesh():
    return plsc.VectorSubcoreMesh(core_axis_name="c", subcore_axis_name="s")


_PI_HI = 3.1415927410125732
_PI_LO = -8.742278012618954e-08
_S1, _S3, _S5, _S7, _S9 = (0.999999997, -0.1666666, 8.33309755e-3,
                           -1.98124848e-4, 2.61290035e-6)
_C0, _C2, _C4, _C6, _C8, _C10 = (1.0, -0.499999995, 4.16666419e-2,
                                 -1.38884323e-3, 2.47637666e-5,
                                 -2.61149497e-7)


def _sincos(x):
    n = jnp.floor(x * (1.0 / jnp.pi) + 0.5)
    r = (x - n * _PI_HI) - n * _PI_LO
    sign = 1.0 - 2.0 * (n - 2.0 * jnp.floor(0.5 * n))
    s = r * r
    sp = ((((_S9 * s + _S7) * s + _S5) * s + _S3) * s + _S1) * r
    cp = ((((_C10 * s + _C8) * s + _C6) * s + _C4) * s + _C2) * s + _C0
    return sign * sp, sign * cp


def _pos_of(ct_blk, wpos_ref):
    return lax.dot_general(ct_blk, wpos_ref[...], (((0,), (0,)), ((), ())),
                           preferred_element_type=jnp.float32)


def _ids_of(ct_blk, invs):
    sx = jnp.floor(ct_blk[0] * invs)
    sy = jnp.floor(ct_blk[1] * invs)
    sz = jnp.floor(ct_blk[2] * invs)
    return ((ct_blk[3] * G1 + sx + 1.0) * G1 + sy + 1.0) * G1 + sz + 1.0


def _k1_body(f_ref, ct_ref, s_ref, wpre_ref, g_ref, b_ref, wpos_ref,
             o_ref, oid_ref):
    h = jnp.dot(f_ref[...], wpre_ref[...], preferred_element_type=jnp.float32)
    mu = jnp.mean(h, axis=1, keepdims=True)
    xc = h - mu
    var = jnp.mean(xc * xc, axis=1, keepdims=True)
    F = xc * lax.rsqrt(var + 1e-6) * g_ref[...] + b_ref[...]
    ct = ct_ref[...]
    pos = _pos_of(ct, wpos_ref)
    sw, cw = _sincos(pos)
    ones = jnp.ones((BR, 1), jnp.float32)
    zer = jnp.zeros((BR, C_AUG - 257), jnp.float32)
    o_ref[...] = jnp.concatenate([F * cw, F * sw, ones, zer], axis=1)
    idsf = _ids_of(ct, s_ref[0, 0])
    oid_ref[...] = idsf.astype(jnp.int32).reshape(BR // 128, 128)


def _k1(feats, ctf, invs, wpreT, gamma, beta, wpos4):
    return pl.pallas_call(
        _k1_body,
        grid=(NBLK,),
        in_specs=[
            pl.BlockSpec((BR, 128), lambda i: (i, 0)),
            pl.BlockSpec((4, BR), lambda i: (0, i)),
            pl.BlockSpec((1, 1), lambda i: (0, 0)),
            pl.BlockSpec((128, 128), lambda i: (0, 0)),
            pl.BlockSpec((1, 128), lambda i: (0, 0)),
            pl.BlockSpec((1, 128), lambda i: (0, 0)),
            pl.BlockSpec((4, 128), lambda i: (0, 0)),
        ],
        out_specs=[
            pl.BlockSpec((BR, C_AUG), lambda i: (i, 0)),
            pl.BlockSpec((BR // 128, 128), lambda i: (i, 0)),
        ],
        out_shape=[
            jax.ShapeDtypeStruct((N_PROC, C_AUG), jnp.float32),
            jax.ShapeDtypeStruct((N_PROC // 128, 128), jnp.int32),
        ],
    )(feats, ctf, invs, wpreT, gamma, beta, wpos4)


def _shift_rows(a, k):
    if k > 0:
        return jnp.concatenate(
            [a[k:], jnp.zeros((k, a.shape[1]), a.dtype)], axis=0)
    return jnp.concatenate(
        [jnp.zeros((-k, a.shape[1]), a.dtype), a[:k]], axis=0)


def _k3_body(g_ref, o_ref, scr):
    x = g_ref[...]
    s = x + _shift_rows(x, 1) + _shift_rows(x, -1)
    s = s + _shift_rows(s, G1) + _shift_rows(s, -G1)
    s0 = s[:PLANE]
    s1 = s[PLANE:]
    out0 = scr[2] + scr[0] + scr[1]
    out1 = scr[0] + scr[1] + s0
    acc = jnp.concatenate([out0, out1], axis=0)
    o_ref[...] = acc[:, :256] / acc[:, 256:257]
    scr[2] = scr[1]
    scr[0] = s0
    scr[1] = s1


def _k3(grid_arr):
    nblk = NPLANES // 2
    return pl.pallas_call(
        _k3_body,
        grid=(nblk + 1,),
        in_specs=[pl.BlockSpec((2 * PLANE, C_AUG),
                               lambda i: (jnp.minimum(i, nblk - 1), 0))],
        out_specs=pl.BlockSpec((2 * PLANE, 256),
                               lambda i: (jnp.maximum(i - 1, 0), 0)),
        out_shape=jax.ShapeDtypeStruct((V_PAD, 256), jnp.float32),
        scratch_shapes=[pltpu.VMEM((3, PLANE, C_AUG), jnp.float32)],
    )(grid_arr)


def _k5_body(g_ref, ct_ref, wpos_ref, o_ref):
    pos = _pos_of(ct_ref[...], wpos_ref)
    sw, cw = _sincos(pos)
    g = g_ref[...]
    o_ref[...] = g[:, :128] * cw + g[:, 128:256] * sw


def _k5(gathered, ctf, wpos4, n):
    return pl.pallas_call(
        _k5_body,
        grid=(NBLK,),
        in_specs=[
            pl.BlockSpec((BR, 256), lambda i: (i, 0)),
            pl.BlockSpec((4, BR), lambda i: (0, i)),
            pl.BlockSpec((4, 128), lambda i: (0, 0)),
        ],
        out_specs=pl.BlockSpec((BR, 128), lambda i: (i, 0)),
        out_shape=jax.ShapeDtypeStruct((n, 128), jnp.float32),
    )(gathered, ctf, wpos4)


def _k2_body(fw_hbm, idx_hbm, zz_hbm, grid_hbm, spg, zbuf, ibuf, vbufa, vbufb,
             obuf, sema, semb):
    c = lax.axis_index("c")
    s = lax.axis_index("s")
    pltpu.sync_copy(zz_hbm, zbuf)
    pltpu.sync_copy(idx_hbm.at[pl.ds(s * 49, 49)], ibuf)
    row0 = s * VT
    p_base = s * PT

    def chunk_iter(jj, carry):
        chunk = jj * 2 + c
        valid = chunk < NCHUNK
        col0 = chunk * 8

        def _src(r):
            return fw_hbm.at[pl.ds(p_base + r * PR, PR), pl.ds(col0, 8)]

        def _start(r, buf, sem):
            pltpu.async_copy(_src(r), buf, sem)

        def _wait(r, buf, sem):
            pltpu.make_async_copy(_src(r), buf, sem).wait()

        def _scat(r, buf):
            for q in range(7):
                pltpu.sync_copy(buf.at[pl.ds(q * 128, 128)],
                                spg.at[ibuf.at[r * 7 + q]], add=True)

        @pl.when(valid)
        def _zero():
            def zr(r, cc):
                pltpu.sync_copy(zbuf, spg.at[pl.ds(row0 + r * SUBZ, SUBZ)])
                return cc
            lax.fori_loop(0, 8, zr, 0)

        plsc.subcore_barrier()

        @pl.when(valid)
        def _scatter():
            _start(0, vbufa, sema)

            def pair(g, cc):
                r = g * 2
                _start(r + 1, vbufb, semb)
                _wait(r, vbufa, sema)
                _scat(r, vbufa)
                _start(r + 2, vbufa, sema)
                _wait(r + 1, vbufb, semb)
                _scat(r + 1, vbufb)
                return cc
            lax.fori_loop(0, SR // 2, pair, 0)
            _wait(SR - 1, vbufa, sema)
            _scat(SR - 1, vbufa)

        plsc.subcore_barrier()

        @pl.when(valid)
        def _readout():
            def ro(r, cc):
                rr = row0 + r * SUBZ
                pltpu.sync_copy(spg.at[pl.ds(rr, SUBZ)], obuf)
                pltpu.sync_copy(obuf,
                                grid_hbm.at[pl.ds(rr, SUBZ), pl.ds(col0, 8)])
                return cc
            lax.fori_loop(0, 8, ro, 0)

        plsc.subcore_barrier()
        return carry

    lax.fori_loop(0, 17, chunk_iter, 0)


@functools.cache
def _k2():
    return pl.kernel(
        _k2_body,
        out_type=jax.ShapeDtypeStruct((V_PAD, C_AUG), jnp.float32),
        mesh=_mesh(),
        compiler_params=pltpu.CompilerParams(use_tc_tiling_on_sc=False),
        scratch_types=[
            pltpu.VMEM_SHARED((V_PAD, 8), jnp.float32),
            pltpu.VMEM((SUBZ, 8), jnp.float32),
            pltpu.VMEM((49, 128), jnp.int32),
            pltpu.VMEM((PR, 8), jnp.float32),
            pltpu.VMEM((PR, 8), jnp.float32),
            pltpu.VMEM((SUBZ, 8), jnp.float32),
            pltpu.SemaphoreType.DMA,
            pltpu.SemaphoreType.DMA,
        ],
    )


def _k4_body(anorm_hbm, idx_hbm, out_hbm, ibuf, rbufa, rbufb, sema, semb):
    c = lax.axis_index("c")
    s = lax.axis_index("s")
    wid = s * 2 + c

    @pl.when(wid < NW4)
    def _work():
        base = wid * PW
        nch = PW // 128
        off = lax.rem(wid * nch, 8)
        abase = pl.multiple_of(wid * nch - off, 8)
        pltpu.sync_copy(idx_hbm.at[pl.ds(abase, 32)], ibuf)

        def _start(k, buf, sem):
            pltpu.async_copy(anorm_hbm.at[ibuf.at[off + k]], buf, sem)

        def _wait(k, buf, sem):
            pltpu.make_async_copy(anorm_hbm.at[ibuf.at[off + k]],
                                  buf, sem).wait()

        def _flush(k, buf):
            pltpu.sync_copy(buf, out_hbm.at[pl.ds(base + k * 128, 128)])

        _start(0, rbufa, sema)

        def pair(g, carry):
            j = g * 2
            _start(j + 1, rbufb, semb)
            _wait(j, rbufa, sema)
            _flush(j, rbufa)
            _start(j + 2, rbufa, sema)
            _wait(j + 1, rbufb, semb)
            _flush(j + 1, rbufb)
            return carry

        lax.fori_loop(0, nch // 2 - 1, pair, 0)
        _start(nch - 1, rbufb, semb)
        _wait(nch - 2, rbufa, sema)
        _flush(nch - 2, rbufa)
        _wait(nch - 1, rbufb, semb)
        _flush(nch - 1, rbufb)


@functools.cache
def _k4():
    return pl.kernel(
        _k4_body,
        out_type=jax.ShapeDtypeStruct((N_PROC, 256), jnp.float32),
        mesh=_mesh(),
        compiler_params=pltpu.CompilerParams(use_tc_tiling_on_sc=True),
        scratch_types=[
            pltpu.VMEM((32, 128), jnp.int32),
            pltpu.VMEM((128, 256), jnp.float32),
            pltpu.VMEM((128, 256), jnp.float32),
            pltpu.SemaphoreType.DMA,
            pltpu.SemaphoreType.DMA,
        ],
    )


def kernel(feats, coords, W_pre, ln_gamma, ln_beta, W_pos, stride):
    n = feats.shape[0]
    ctf = coords.T.astype(jnp.float32)
    padcol = jnp.tile(jnp.array([[0.0], [0.0], [0.0], [4.0]], jnp.float32),
                      (1, N_PROC - n))
    ctf = jnp.concatenate([ctf, padcol], axis=1)
    invs = jnp.reshape(1.0 / jnp.asarray(stride, jnp.float32), (1, 1))
    wpreT = W_pre.T
    wpos4 = jnp.pad(W_pos.T.astype(jnp.float32), ((0, 1), (0, 0)))
    gamma = ln_gamma.reshape(1, 128)
    beta = ln_beta.reshape(1, 128)
    zz = jnp.zeros((SUBZ, 8), jnp.float32)

    fw_aug, idx2 = _k1(feats, ctf, invs, wpreT, gamma, beta, wpos4)
    grid_arr = _k2()(fw_aug, idx2, zz)
    anorm = _k3(grid_arr)
    gathered = _k4()(anorm, idx2)
    return _k5(gathered, ctf, wpos4, n)


# speedup vs baseline: 145.6683x; 1.3499x over previous
import functools

import jax
import jax.numpy as jnp
from jax import lax
from jax.experimental import pallas as pl
from jax.experimental.pallas import tpu as pltpu
from jax.experimental.pallas import tpu_sc as plsc

G1 = 34
PLANE = G1 * G1
NPLANES = 4 * G1
V = NPLANES * PLANE
V_PAD = 158464
VT = V_PAD // 16
SUBZ = 1238
DUMMY = 158407

NCHUNK = 33

N_PROC = 100352
BR = 1024
NBLK = N_PROC // BR
PT = N_PROC // 16
SR = 7
PR = 896
NW4 = 28
PW = N_PROC // NW4


@functools.cache
def _mesh():
    return plsc.VectorSubcoreMesh(core_axis_name="c", subcore_axis_name="s")


_PI_HI = 3.1415927410125732
_PI_LO = -8.742278012618954e-08
_S1, _S3, _S5, _S7, _S9 = (0.999999997, -0.1666666, 8.33309755e-3,
                           -1.98124848e-4, 2.61290035e-6)
_C0, _C2, _C4, _C6, _C8, _C10 = (1.0, -0.499999995, 4.16666419e-2,
                                 -1.38884323e-3, 2.47637666e-5,
                                 -2.61149497e-7)


def _sincos(x):
    n = jnp.floor(x * (1.0 / jnp.pi) + 0.5)
    r = (x - n * _PI_HI) - n * _PI_LO
    sign = 1.0 - 2.0 * (n - 2.0 * jnp.floor(0.5 * n))
    s = r * r
    sp = ((((_S9 * s + _S7) * s + _S5) * s + _S3) * s + _S1) * r
    cp = ((((_C10 * s + _C8) * s + _C6) * s + _C4) * s + _C2) * s + _C0
    return sign * sp, sign * cp


def _pos_of(ct_blk, wpos_ref):
    return lax.dot_general(ct_blk, wpos_ref[...], (((0,), (0,)), ((), ())),
                           preferred_element_type=jnp.float32)


def _ids_of(ct_blk, invs):
    sx = jnp.floor(ct_blk[0] * invs)
    sy = jnp.floor(ct_blk[1] * invs)
    sz = jnp.floor(ct_blk[2] * invs)
    return ((ct_blk[3] * G1 + sx + 1.0) * G1 + sy + 1.0) * G1 + sz + 1.0


def _k1_body(f_ref, ct_ref, s_ref, wpre_ref, g_ref, b_ref, wpos_ref,
             o_ref, oid_ref):
    h = jnp.dot(f_ref[...], wpre_ref[...], preferred_element_type=jnp.float32)
    mu = jnp.mean(h, axis=1, keepdims=True)
    xc = h - mu
    var = jnp.mean(xc * xc, axis=1, keepdims=True)
    F = xc * lax.rsqrt(var + 1e-6) * g_ref[...] + b_ref[...]
    ct = ct_ref[...]
    pos = _pos_of(ct, wpos_ref)
    sw, cw = _sincos(pos)
    o_ref[...] = jnp.stack([F * cw, F * sw], axis=0)
    idsf = _ids_of(ct, s_ref[0, 0])
    oid_ref[...] = idsf.astype(jnp.int32).reshape(BR // 128, 128)


def _k1(feats, ctf, invs, wpreT, gamma, beta, wpos4):
    return pl.pallas_call(
        _k1_body,
        grid=(NBLK,),
        in_specs=[
            pl.BlockSpec((BR, 128), lambda i: (i, 0)),
            pl.BlockSpec((4, BR), lambda i: (0, i)),
            pl.BlockSpec((1, 1), lambda i: (0, 0)),
            pl.BlockSpec((128, 128), lambda i: (0, 0)),
            pl.BlockSpec((1, 128), lambda i: (0, 0)),
            pl.BlockSpec((1, 128), lambda i: (0, 0)),
            pl.BlockSpec((4, 128), lambda i: (0, 0)),
        ],
        out_specs=[
            pl.BlockSpec((2, BR, 128), lambda i: (0, i, 0)),
            pl.BlockSpec((BR // 128, 128), lambda i: (i, 0)),
        ],
        out_shape=[
            jax.ShapeDtypeStruct((2, N_PROC, 128), jnp.float32),
            jax.ShapeDtypeStruct((N_PROC // 128, 128), jnp.int32),
        ],
    )(feats, ctf, invs, wpreT, gamma, beta, wpos4)


def _shift1(a, k):
    z = list(a.shape)
    z[-2] = abs(k)
    zer = jnp.zeros(z, a.dtype)
    if k > 0:
        return jnp.concatenate([a[..., k:, :], zer], axis=-2)
    return jnp.concatenate([zer, a[..., :k, :]], axis=-2)


def _box_zy(x):
    s = x + _shift1(x, 1) + _shift1(x, -1)
    return s + _shift1(s, G1) + _shift1(s, -G1)


def _k3_body(g_ref, c_ref, o_ref, scrf, scrc):
    s = _box_zy(g_ref[...])
    t = _box_zy(c_ref[...])
    s0 = s[:, :PLANE]
    s1 = s[:, PLANE:]
    t0 = t[:PLANE]
    t1 = t[PLANE:]
    outf0 = scrf[2] + scrf[0] + scrf[1]
    outf1 = scrf[0] + scrf[1] + s0
    outc0 = scrc[2] + scrc[0] + scrc[1]
    outc1 = scrc[0] + scrc[1] + t0
    a0 = outf0 / outc0[None, :, 0:1]
    a1 = outf1 / outc1[None, :, 0:1]
    o_ref[...] = jnp.concatenate([a0, a1], axis=1)
    scrf[2] = scrf[1]
    scrf[0] = s0
    scrf[1] = s1
    scrc[2] = scrc[1]
    scrc[0] = t0
    scrc[1] = t1


def _k3(grid3, gc):
    nblk = NPLANES // 2
    return pl.pallas_call(
        _k3_body,
        grid=(nblk + 1,),
        in_specs=[
            pl.BlockSpec((2, 2 * PLANE, 128),
                         lambda i: (0, jnp.minimum(i, nblk - 1), 0)),
            pl.BlockSpec((2 * PLANE, 8),
                         lambda i: (jnp.minimum(i, nblk - 1), 0)),
        ],
        out_specs=pl.BlockSpec((2, 2 * PLANE, 128),
                               lambda i: (0, jnp.maximum(i - 1, 0), 0)),
        out_shape=jax.ShapeDtypeStruct((2, V_PAD, 128), jnp.float32),
        scratch_shapes=[
            pltpu.VMEM((3, 2, PLANE, 128), jnp.float32),
            pltpu.VMEM((3, PLANE, 8), jnp.float32),
        ],
    )(grid3, gc)


def _k5_body(lo_ref, hi_ref, ct_ref, wpos_ref, o_ref):
    pos = _pos_of(ct_ref[...], wpos_ref)
    sw, cw = _sincos(pos)
    o_ref[...] = lo_ref[...] * cw + hi_ref[...] * sw


def _k5(lo, hi, ctf, wpos4, n):
    return pl.pallas_call(
        _k5_body,
        grid=(NBLK,),
        in_specs=[
            pl.BlockSpec((BR, 128), lambda i: (i, 0)),
            pl.BlockSpec((BR, 128), lambda i: (i, 0)),
            pl.BlockSpec((4, BR), lambda i: (0, i)),
            pl.BlockSpec((4, 128), lambda i: (0, 0)),
        ],
        out_specs=pl.BlockSpec((BR, 128), lambda i: (i, 0)),
        out_shape=jax.ShapeDtypeStruct((n, 128), jnp.float32),
    )(lo, hi, ctf, wpos4)


def _k2_body(fw_hbm, idx_hbm, zz_hbm, ones_hbm, grid_hbm, gc_hbm,
             spg, zbuf, ibuf, vbufa, vbufb, onesbuf, obuf, sema, semb):
    c = lax.axis_index("c")
    s = lax.axis_index("s")
    pltpu.sync_copy(zz_hbm, zbuf)
    pltpu.sync_copy(ones_hbm, onesbuf)
    pltpu.sync_copy(idx_hbm.at[pl.ds(s * 49, 49)], ibuf)
    row0 = s * VT
    p_base = s * PT

    def chunk_iter(jj, carry):
        chunk = jj * 2 + c
        valid = chunk < NCHUNK
        is_cnt = chunk == NCHUNK - 1
        half = chunk // 16
        col0 = (chunk % 16) * 8

        def _src(r):
            return fw_hbm.at[half, pl.ds(p_base + r * PR, PR),
                             pl.ds(col0, 8)]

        def _start(r, buf, sem):
            pltpu.async_copy(_src(r), buf, sem)

        def _wait(r, buf, sem):
            pltpu.make_async_copy(_src(r), buf, sem).wait()

        def _scat(r, buf):
            for q in range(7):
                pltpu.sync_copy(buf.at[pl.ds(q * 128, 128)],
                                spg.at[ibuf.at[r * 7 + q]], add=True)

        @pl.when(valid)
        def _zero():
            def zr(r, cc):
                pltpu.sync_copy(zbuf, spg.at[pl.ds(row0 + r * SUBZ, SUBZ)])
                return cc
            lax.fori_loop(0, 8, zr, 0)

        plsc.subcore_barrier()

        @pl.when(valid & jnp.logical_not(is_cnt))
        def _scatter():
            _start(0, vbufa, sema)

            def pair(g, cc):
                r = g * 2
                _start(r + 1, vbufb, semb)
                _wait(r, vbufa, sema)
                _scat(r, vbufa)
                _start(r + 2, vbufa, sema)
                _wait(r + 1, vbufb, semb)
                _scat(r + 1, vbufb)
                return cc
            lax.fori_loop(0, SR // 2, pair, 0)
            _wait(SR - 1, vbufa, sema)
            _scat(SR - 1, vbufa)

        @pl.when(is_cnt)
        def _scatter_ones():
            def sc1(q, cc):
                pltpu.sync_copy(onesbuf, spg.at[ibuf.at[q]], add=True)
                return cc
            lax.fori_loop(0, 49, sc1, 0)

        plsc.subcore_barrier()

        @pl.when(valid & jnp.logical_not(is_cnt))
        def _readout():
            def ro(r, cc):
                rr = row0 + r * SUBZ
                pltpu.sync_copy(spg.at[pl.ds(rr, SUBZ)], obuf)
                pltpu.sync_copy(obuf,
                                grid_hbm.at[half, pl.ds(rr, SUBZ),
                                            pl.ds(col0, 8)])
                return cc
            lax.fori_loop(0, 8, ro, 0)

        @pl.when(is_cnt)
        def _readout_cnt():
            def ro(r, cc):
                rr = row0 + r * SUBZ
                pltpu.sync_copy(spg.at[pl.ds(rr, SUBZ)], obuf)
                pltpu.sync_copy(obuf, gc_hbm.at[pl.ds(rr, SUBZ)])
                return cc
            lax.fori_loop(0, 8, ro, 0)

        plsc.subcore_barrier()
        return carry

    lax.fori_loop(0, 17, chunk_iter, 0)


@functools.cache
def _k2():
    return pl.kernel(
        _k2_body,
        out_type=[
            jax.ShapeDtypeStruct((2, V_PAD, 128), jnp.float32),
            jax.ShapeDtypeStruct((V_PAD, 8), jnp.float32),
        ],
        mesh=_mesh(),
        compiler_params=pltpu.CompilerParams(use_tc_tiling_on_sc=False),
        scratch_types=[
            pltpu.VMEM_SHARED((V_PAD, 8), jnp.float32),
            pltpu.VMEM((SUBZ, 8), jnp.float32),
            pltpu.VMEM((49, 128), jnp.int32),
            pltpu.VMEM((PR, 8), jnp.float32),
            pltpu.VMEM((PR, 8), jnp.float32),
            pltpu.VMEM((128, 8), jnp.float32),
            pltpu.VMEM((SUBZ, 8), jnp.float32),
            pltpu.SemaphoreType.DMA,
            pltpu.SemaphoreType.DMA,
        ],
    )


def _k4_body(anorm_hbm, idx_hbm, lo_hbm, hi_hbm,
             ibuf, ibuf2, rbufa, rbufb, sema, semb):
    c = lax.axis_index("c")
    s = lax.axis_index("s")
    wid = s * 2 + c

    @pl.when(wid < NW4)
    def _work():
        base = wid * PW
        nch = PW // 128
        off = lax.rem(wid * nch, 8)
        abase = pl.multiple_of(wid * nch - off, 8)
        pltpu.sync_copy(idx_hbm.at[pl.ds(abase, 32)], ibuf)

        def addv(i, cc):
            for q in range(8):
                ibuf2[i, pl.ds(q * 16, 16)] = (
                    ibuf[i, pl.ds(q * 16, 16)] + V_PAD)
            return cc
        lax.fori_loop(0, 32, addv, 0)

        def _start(k, buf, sem):
            pltpu.async_copy(anorm_hbm.at[ibuf.at[off + k]], buf.at[0], sem)
            pltpu.async_copy(anorm_hbm.at[ibuf2.at[off + k]], buf.at[1], sem)

        def _wait(k, buf, sem):
            pltpu.make_async_copy(anorm_hbm.at[ibuf.at[off + k]],
                                  buf.at[0], sem).wait()
            pltpu.make_async_copy(anorm_hbm.at[ibuf2.at[off + k]],
                                  buf.at[1], sem).wait()

        def _flush(k, buf):
            pltpu.sync_copy(buf.at[0], lo_hbm.at[pl.ds(base + k * 128, 128)])
            pltpu.sync_copy(buf.at[1], hi_hbm.at[pl.ds(base + k * 128, 128)])

        _start(0, rbufa, sema)

        def pair(g, carry):
            j = g * 2
            _start(j + 1, rbufb, semb)
            _wait(j, rbufa, sema)
            _flush(j, rbufa)
            _start(j + 2, rbufa, sema)
            _wait(j + 1, rbufb, semb)
            _flush(j + 1, rbufb)
            return carry

        lax.fori_loop(0, nch // 2 - 1, pair, 0)
        _start(nch - 1, rbufb, semb)
        _wait(nch - 2, rbufa, sema)
        _flush(nch - 2, rbufa)
        _wait(nch - 1, rbufb, semb)
        _flush(nch - 1, rbufb)


@functools.cache
def _k4():
    return pl.kernel(
        _k4_body,
        out_type=[
            jax.ShapeDtypeStruct((N_PROC, 128), jnp.float32),
            jax.ShapeDtypeStruct((N_PROC, 128), jnp.float32),
        ],
        mesh=_mesh(),
        compiler_params=pltpu.CompilerParams(use_tc_tiling_on_sc=True),
        scratch_types=[
            pltpu.VMEM((32, 128), jnp.int32),
            pltpu.VMEM((32, 128), jnp.int32),
            pltpu.VMEM((2, 128, 128), jnp.float32),
            pltpu.VMEM((2, 128, 128), jnp.float32),
            pltpu.SemaphoreType.DMA,
            pltpu.SemaphoreType.DMA,
        ],
    )


def kernel(feats, coords, W_pre, ln_gamma, ln_beta, W_pos, stride):
    n = feats.shape[0]
    ctf = coords.T.astype(jnp.float32)
    padcol = jnp.tile(jnp.array([[0.0], [0.0], [0.0], [4.0]], jnp.float32),
                      (1, N_PROC - n))
    ctf = jnp.concatenate([ctf, padcol], axis=1)
    invs = jnp.reshape(1.0 / jnp.asarray(stride, jnp.float32), (1, 1))
    wpreT = W_pre.T
    wpos4 = jnp.pad(W_pos.T.astype(jnp.float32), ((0, 1), (0, 0)))
    gamma = ln_gamma.reshape(1, 128)
    beta = ln_beta.reshape(1, 128)
    zz = jnp.zeros((SUBZ, 8), jnp.float32)
    cones = jnp.zeros((128, 8), jnp.float32).at[:, 0].set(1.0)

    fw, idx2 = _k1(feats, ctf, invs, wpreT, gamma, beta, wpos4)
    grid3, gc = _k2()(fw, idx2, zz, cones)
    anorm3 = _k3(grid3, gc)
    anorm2 = anorm3.reshape(2 * V_PAD, 128)
    lo, hi = _k4()(anorm2, idx2)
    return _k5(lo, hi, ctf, wpos4, n)


# speedup vs baseline: 147.9371x; 1.0156x over previous
import functools

import jax
import jax.numpy as jnp
from jax import lax
from jax.experimental import pallas as pl
from jax.experimental.pallas import tpu as pltpu
from jax.experimental.pallas import tpu_sc as plsc

G1 = 34
PLANE = G1 * G1
NPLANES = 4 * G1
V = NPLANES * PLANE
V_PAD = 158464
VT = V_PAD // 16
SUBZ = 1238
DUMMY = 158407

NCHUNK = 33

N_PROC = 100352
BR = 1024
NBLK = N_PROC // BR
PT = N_PROC // 16
SR = 7
PR = 896
NW4 = 28
PW = N_PROC // NW4


@functools.cache
def _mesh():
    return plsc.VectorSubcoreMesh(core_axis_name="c", subcore_axis_name="s")


_PI_HI = 3.1415927410125732
_PI_LO = -8.742278012618954e-08
_S1, _S3, _S5, _S7, _S9 = (0.999999997, -0.1666666, 8.33309755e-3,
                           -1.98124848e-4, 2.61290035e-6)
_C0, _C2, _C4, _C6, _C8, _C10 = (1.0, -0.499999995, 4.16666419e-2,
                                 -1.38884323e-3, 2.47637666e-5,
                                 -2.61149497e-7)


def _sincos(x):
    n = jnp.floor(x * (1.0 / jnp.pi) + 0.5)
    r = (x - n * _PI_HI) - n * _PI_LO
    sign = 1.0 - 2.0 * (n - 2.0 * jnp.floor(0.5 * n))
    s = r * r
    sp = ((((_S9 * s + _S7) * s + _S5) * s + _S3) * s + _S1) * r
    cp = ((((_C10 * s + _C8) * s + _C6) * s + _C4) * s + _C2) * s + _C0
    return sign * sp, sign * cp


def _pos_of(ct_blk, wpos_ref):
    return lax.dot_general(ct_blk, wpos_ref[...], (((0,), (0,)), ((), ())),
                           preferred_element_type=jnp.float32)


def _ids_of(ct_blk, invs):
    sx = jnp.floor(ct_blk[0] * invs)
    sy = jnp.floor(ct_blk[1] * invs)
    sz = jnp.floor(ct_blk[2] * invs)
    return ((ct_blk[3] * G1 + sx + 1.0) * G1 + sy + 1.0) * G1 + sz + 1.0


def _k1_body(f_ref, ct_ref, s_ref, wpre_ref, g_ref, b_ref, wpos_ref,
             o_ref, oid_ref):
    h = jnp.dot(f_ref[...], wpre_ref[...], preferred_element_type=jnp.float32)
    mu = jnp.mean(h, axis=1, keepdims=True)
    xc = h - mu
    var = jnp.mean(xc * xc, axis=1, keepdims=True)
    F = xc * lax.rsqrt(var + 1e-6) * g_ref[...] + b_ref[...]
    ct = ct_ref[...]
    pos = _pos_of(ct, wpos_ref)
    sw, cw = _sincos(pos)
    o_ref[...] = jnp.stack([F * cw, F * sw], axis=0)
    idsf = _ids_of(ct, s_ref[0, 0])
    oid_ref[...] = idsf.astype(jnp.int32).reshape(BR // 128, 128)


def _k1(feats, ctf, invs, wpreT, gamma, beta, wpos4):
    return pl.pallas_call(
        _k1_body,
        grid=(NBLK,),
        in_specs=[
            pl.BlockSpec((BR, 128), lambda i: (i, 0)),
            pl.BlockSpec((4, BR), lambda i: (0, i)),
            pl.BlockSpec((1, 1), lambda i: (0, 0)),
            pl.BlockSpec((128, 128), lambda i: (0, 0)),
            pl.BlockSpec((1, 128), lambda i: (0, 0)),
            pl.BlockSpec((1, 128), lambda i: (0, 0)),
            pl.BlockSpec((4, 128), lambda i: (0, 0)),
        ],
        out_specs=[
            pl.BlockSpec((2, BR, 128), lambda i: (0, i, 0)),
            pl.BlockSpec((BR // 128, 128), lambda i: (i, 0)),
        ],
        out_shape=[
            jax.ShapeDtypeStruct((2, N_PROC, 128), jnp.float32),
            jax.ShapeDtypeStruct((N_PROC // 128, 128), jnp.int32),
        ],
    )(feats, ctf, invs, wpreT, gamma, beta, wpos4)


def _shift1(a, k):
    z = list(a.shape)
    z[-2] = abs(k)
    zer = jnp.zeros(z, a.dtype)
    if k > 0:
        return jnp.concatenate([a[..., k:, :], zer], axis=-2)
    return jnp.concatenate([zer, a[..., :k, :]], axis=-2)


def _box_zy(x):
    s = x + _shift1(x, 1) + _shift1(x, -1)
    return s + _shift1(s, G1) + _shift1(s, -G1)


def _k3_body(g_ref, c_ref, o_ref, scrf, scrc):
    s = _box_zy(g_ref[...])
    t = _box_zy(c_ref[...])
    s0 = s[:, :PLANE]
    s1 = s[:, PLANE:]
    t0 = t[:PLANE]
    t1 = t[PLANE:]
    outf0 = scrf[2] + scrf[0] + scrf[1]
    outf1 = scrf[0] + scrf[1] + s0
    outc0 = scrc[2] + scrc[0] + scrc[1]
    outc1 = scrc[0] + scrc[1] + t0
    a0 = outf0 / outc0[None, :, 0:1]
    a1 = outf1 / outc1[None, :, 0:1]
    o_ref[...] = jnp.concatenate([a0, a1], axis=1)
    scrf[2] = scrf[1]
    scrf[0] = s0
    scrf[1] = s1
    scrc[2] = scrc[1]
    scrc[0] = t0
    scrc[1] = t1


def _k3(grid3, gc):
    nblk = NPLANES // 2
    return pl.pallas_call(
        _k3_body,
        grid=(nblk + 1,),
        in_specs=[
            pl.BlockSpec((2, 2 * PLANE, 128),
                         lambda i: (0, jnp.minimum(i, nblk - 1), 0)),
            pl.BlockSpec((2 * PLANE, 8),
                         lambda i: (jnp.minimum(i, nblk - 1), 0)),
        ],
        out_specs=pl.BlockSpec((2, 2 * PLANE, 128),
                               lambda i: (0, jnp.maximum(i - 1, 0), 0)),
        out_shape=jax.ShapeDtypeStruct((2, V_PAD, 128), jnp.float32),
        scratch_shapes=[
            pltpu.VMEM((3, 2, PLANE, 128), jnp.float32),
            pltpu.VMEM((3, PLANE, 8), jnp.float32),
        ],
    )(grid3, gc)


def _k5_body(lo_ref, hi_ref, ct_ref, wpos_ref, o_ref):
    pos = _pos_of(ct_ref[...], wpos_ref)
    sw, cw = _sincos(pos)
    o_ref[...] = lo_ref[...] * cw + hi_ref[...] * sw


def _k5(lo, hi, ctf, wpos4, n):
    return pl.pallas_call(
        _k5_body,
        grid=(NBLK,),
        in_specs=[
            pl.BlockSpec((BR, 128), lambda i: (i, 0)),
            pl.BlockSpec((BR, 128), lambda i: (i, 0)),
            pl.BlockSpec((4, BR), lambda i: (0, i)),
            pl.BlockSpec((4, 128), lambda i: (0, 0)),
        ],
        out_specs=pl.BlockSpec((BR, 128), lambda i: (i, 0)),
        out_shape=jax.ShapeDtypeStruct((n, 128), jnp.float32),
    )(lo, hi, ctf, wpos4)


def _k2_body(fw_hbm, idx_hbm, zz_hbm, ones_hbm, grid_hbm, gc_hbm,
             spg, zbuf, ibuf, vbufa, vbufb, onesbuf, obuf, sema, semb, sems):
    c = lax.axis_index("c")
    s = lax.axis_index("s")
    pltpu.sync_copy(zz_hbm, zbuf)
    pltpu.sync_copy(ones_hbm, onesbuf)
    pltpu.sync_copy(idx_hbm.at[pl.ds(s * 49, 49)], ibuf)
    row0 = s * VT
    p_base = s * PT

    def chunk_iter(jj, carry):
        chunk = jj * 2 + c
        valid = chunk < NCHUNK
        is_cnt = chunk == NCHUNK - 1
        half = chunk // 16
        col0 = (chunk % 16) * 8

        def _src(r):
            return fw_hbm.at[half, pl.ds(p_base + r * PR, PR),
                             pl.ds(col0, 8)]

        def _start(r, buf, sem):
            pltpu.async_copy(_src(r), buf, sem)

        def _wait(r, buf, sem):
            pltpu.make_async_copy(_src(r), buf, sem).wait()

        def _scat(r, buf):
            for q in range(7):
                pltpu.async_copy(buf.at[pl.ds(q * 128, 128)],
                                 spg.at[ibuf.at[r * 7 + q]], sems, add=True)
            for q in range(7):
                pltpu.make_async_copy(buf.at[pl.ds(q * 128, 128)],
                                      spg.at[ibuf.at[r * 7 + q]],
                                      sems).wait()

        @pl.when(valid)
        def _zero():
            def zr(r, cc):
                pltpu.sync_copy(zbuf, spg.at[pl.ds(row0 + r * SUBZ, SUBZ)])
                return cc
            lax.fori_loop(0, 8, zr, 0)

        plsc.subcore_barrier()

        @pl.when(valid & jnp.logical_not(is_cnt))
        def _scatter():
            _start(0, vbufa, sema)

            def pair(g, cc):
                r = g * 2
                _start(r + 1, vbufb, semb)
                _wait(r, vbufa, sema)
                _scat(r, vbufa)
                _start(r + 2, vbufa, sema)
                _wait(r + 1, vbufb, semb)
                _scat(r + 1, vbufb)
                return cc
            lax.fori_loop(0, SR // 2, pair, 0)
            _wait(SR - 1, vbufa, sema)
            _scat(SR - 1, vbufa)

        @pl.when(is_cnt)
        def _scatter_ones():
            def sc1(r, cc):
                for q in range(7):
                    pltpu.async_copy(onesbuf, spg.at[ibuf.at[r * 7 + q]],
                                     sems, add=True)
                for q in range(7):
                    pltpu.make_async_copy(onesbuf,
                                          spg.at[ibuf.at[r * 7 + q]],
                                          sems).wait()
                return cc
            lax.fori_loop(0, 7, sc1, 0)

        plsc.subcore_barrier()

        @pl.when(valid & jnp.logical_not(is_cnt))
        def _readout():
            def ro(r, cc):
                rr = row0 + r * SUBZ
                pltpu.sync_copy(spg.at[pl.ds(rr, SUBZ)], obuf)
                pltpu.sync_copy(obuf,
                                grid_hbm.at[half, pl.ds(rr, SUBZ),
                                            pl.ds(col0, 8)])
                return cc
            lax.fori_loop(0, 8, ro, 0)

        @pl.when(is_cnt)
        def _readout_cnt():
            def ro(r, cc):
                rr = row0 + r * SUBZ
                pltpu.sync_copy(spg.at[pl.ds(rr, SUBZ)], obuf)
                pltpu.sync_copy(obuf, gc_hbm.at[pl.ds(rr, SUBZ)])
                return cc
            lax.fori_loop(0, 8, ro, 0)

        plsc.subcore_barrier()
        return carry

    lax.fori_loop(0, 17, chunk_iter, 0)


@functools.cache
def _k2():
    return pl.kernel(
        _k2_body,
        out_type=[
            jax.ShapeDtypeStruct((2, V_PAD, 128), jnp.float32),
            jax.ShapeDtypeStruct((V_PAD, 8), jnp.float32),
        ],
        mesh=_mesh(),
        compiler_params=pltpu.CompilerParams(use_tc_tiling_on_sc=False),
        scratch_types=[
            pltpu.VMEM_SHARED((V_PAD, 8), jnp.float32),
            pltpu.VMEM((SUBZ, 8), jnp.float32),
            pltpu.VMEM((49, 128), jnp.int32),
            pltpu.VMEM((PR, 8), jnp.float32),
            pltpu.VMEM((PR, 8), jnp.float32),
            pltpu.VMEM((128, 8), jnp.float32),
            pltpu.VMEM((SUBZ, 8), jnp.float32),
            pltpu.SemaphoreType.DMA,
            pltpu.SemaphoreType.DMA,
            pltpu.SemaphoreType.DMA,
        ],
    )


def _k4_body(anorm_hbm, idx_hbm, lo_hbm, hi_hbm,
             ibuf, ibuf2, rbufa, rbufb, sema, semb):
    c = lax.axis_index("c")
    s = lax.axis_index("s")
    wid = s * 2 + c

    @pl.when(wid < NW4)
    def _work():
        base = wid * PW
        nch = PW // 128
        off = lax.rem(wid * nch, 8)
        abase = pl.multiple_of(wid * nch - off, 8)
        pltpu.sync_copy(idx_hbm.at[pl.ds(abase, 32)], ibuf)

        def addv(i, cc):
            for q in range(8):
                ibuf2[i, pl.ds(q * 16, 16)] = (
                    ibuf[i, pl.ds(q * 16, 16)] + V_PAD)
            return cc
        lax.fori_loop(0, 32, addv, 0)

        def _start(k, buf, sem):
            pltpu.async_copy(anorm_hbm.at[ibuf.at[off + k]], buf.at[0], sem)
            pltpu.async_copy(anorm_hbm.at[ibuf2.at[off + k]], buf.at[1], sem)

        def _wait(k, buf, sem):
            pltpu.make_async_copy(anorm_hbm.at[ibuf.at[off + k]],
                                  buf.at[0], sem).wait()
            pltpu.make_async_copy(anorm_hbm.at[ibuf2.at[off + k]],
                                  buf.at[1], sem).wait()

        def _flush(k, buf):
            pltpu.sync_copy(buf.at[0], lo_hbm.at[pl.ds(base + k * 128, 128)])
            pltpu.sync_copy(buf.at[1], hi_hbm.at[pl.ds(base + k * 128, 128)])

        _start(0, rbufa, sema)

        def pair(g, carry):
            j = g * 2
            _start(j + 1, rbufb, semb)
            _wait(j, rbufa, sema)
            _flush(j, rbufa)
            _start(j + 2, rbufa, sema)
            _wait(j + 1, rbufb, semb)
            _flush(j + 1, rbufb)
            return carry

        lax.fori_loop(0, nch // 2 - 1, pair, 0)
        _start(nch - 1, rbufb, semb)
        _wait(nch - 2, rbufa, sema)
        _flush(nch - 2, rbufa)
        _wait(nch - 1, rbufb, semb)
        _flush(nch - 1, rbufb)


@functools.cache
def _k4():
    return pl.kernel(
        _k4_body,
        out_type=[
            jax.ShapeDtypeStruct((N_PROC, 128), jnp.float32),
            jax.ShapeDtypeStruct((N_PROC, 128), jnp.float32),
        ],
        mesh=_mesh(),
        compiler_params=pltpu.CompilerParams(use_tc_tiling_on_sc=True),
        scratch_types=[
            pltpu.VMEM((32, 128), jnp.int32),
            pltpu.VMEM((32, 128), jnp.int32),
            pltpu.VMEM((2, 128, 128), jnp.float32),
            pltpu.VMEM((2, 128, 128), jnp.float32),
            pltpu.SemaphoreType.DMA,
            pltpu.SemaphoreType.DMA,
        ],
    )


def kernel(feats, coords, W_pre, ln_gamma, ln_beta, W_pos, stride):
    n = feats.shape[0]
    ctf = coords.T.astype(jnp.float32)
    padcol = jnp.tile(jnp.array([[0.0], [0.0], [0.0], [4.0]], jnp.float32),
                      (1, N_PROC - n))
    ctf = jnp.concatenate([ctf, padcol], axis=1)
    invs = jnp.reshape(1.0 / jnp.asarray(stride, jnp.float32), (1, 1))
    wpreT = W_pre.T
    wpos4 = jnp.pad(W_pos.T.astype(jnp.float32), ((0, 1), (0, 0)))
    gamma = ln_gamma.reshape(1, 128)
    beta = ln_beta.reshape(1, 128)
    zz = jnp.zeros((SUBZ, 8), jnp.float32)
    cones = jnp.zeros((128, 8), jnp.float32).at[:, 0].set(1.0)

    fw, idx2 = _k1(feats, ctf, invs, wpreT, gamma, beta, wpos4)
    grid3, gc = _k2()(fw, idx2, zz, cones)
    anorm3 = _k3(grid3, gc)
    anorm2 = anorm3.reshape(2 * V_PAD, 128)
    lo, hi = _k4()(anorm2, idx2)
    return _k5(lo, hi, ctf, wpos4, n)


# speedup vs baseline: 155.2512x; 1.0494x over previous
import functools

import jax
import jax.numpy as jnp
from jax import lax
from jax.experimental import pallas as pl
from jax.experimental.pallas import tpu as pltpu
from jax.experimental.pallas import tpu_sc as plsc

G1 = 34
PLANE = G1 * G1
NPLANES = 4 * G1
V = NPLANES * PLANE
V_PAD = 158464
VT = V_PAD // 16
SUBZ = 1238
DUMMY = 158407

NCHUNK = 33

N_PROC = 100352
BR = 1024
NBLK = N_PROC // BR
PT = N_PROC // 16
SR = 7
PR = 896
NW4 = 28
PW = N_PROC // NW4


@functools.cache
def _mesh():
    return plsc.VectorSubcoreMesh(core_axis_name="c", subcore_axis_name="s")


_PI_HI = 3.1415927410125732
_PI_LO = -8.742278012618954e-08
_S1, _S3, _S5, _S7, _S9 = (0.999999997, -0.1666666, 8.33309755e-3,
                           -1.98124848e-4, 2.61290035e-6)
_C0, _C2, _C4, _C6, _C8, _C10 = (1.0, -0.499999995, 4.16666419e-2,
                                 -1.38884323e-3, 2.47637666e-5,
                                 -2.61149497e-7)


def _sincos(x):
    n = jnp.floor(x * (1.0 / jnp.pi) + 0.5)
    r = (x - n * _PI_HI) - n * _PI_LO
    sign = 1.0 - 2.0 * (n - 2.0 * jnp.floor(0.5 * n))
    s = r * r
    sp = ((((_S9 * s + _S7) * s + _S5) * s + _S3) * s + _S1) * r
    cp = ((((_C10 * s + _C8) * s + _C6) * s + _C4) * s + _C2) * s + _C0
    return sign * sp, sign * cp


def _pos_of(ct_blk, wpos_ref):
    return lax.dot_general(ct_blk, wpos_ref[...], (((0,), (0,)), ((), ())),
                           preferred_element_type=jnp.float32)


def _ids_of(ct_blk, invs):
    sx = jnp.floor(ct_blk[0] * invs)
    sy = jnp.floor(ct_blk[1] * invs)
    sz = jnp.floor(ct_blk[2] * invs)
    return ((ct_blk[3] * G1 + sx + 1.0) * G1 + sy + 1.0) * G1 + sz + 1.0


def _k1_body(f_ref, ct_ref, s_ref, wpre_ref, g_ref, b_ref, wpos_ref,
             o_ref, oid_ref):
    h = jnp.dot(f_ref[...], wpre_ref[...], preferred_element_type=jnp.float32)
    mu = jnp.mean(h, axis=1, keepdims=True)
    xc = h - mu
    var = jnp.mean(xc * xc, axis=1, keepdims=True)
    F = xc * lax.rsqrt(var + 1e-6) * g_ref[...] + b_ref[...]
    ct = ct_ref[...]
    pos = _pos_of(ct, wpos_ref)
    sw, cw = _sincos(pos)
    o_ref[...] = jnp.stack([F * cw, F * sw], axis=0)
    idsf = _ids_of(ct, s_ref[0, 0])
    oid_ref[...] = idsf.astype(jnp.int32).reshape(BR // 128, 128)


def _k1(feats, ctf, invs, wpreT, gamma, beta, wpos4):
    return pl.pallas_call(
        _k1_body,
        grid=(NBLK,),
        in_specs=[
            pl.BlockSpec((BR, 128), lambda i: (i, 0)),
            pl.BlockSpec((4, BR), lambda i: (0, i)),
            pl.BlockSpec((1, 1), lambda i: (0, 0)),
            pl.BlockSpec((128, 128), lambda i: (0, 0)),
            pl.BlockSpec((1, 128), lambda i: (0, 0)),
            pl.BlockSpec((1, 128), lambda i: (0, 0)),
            pl.BlockSpec((4, 128), lambda i: (0, 0)),
        ],
        out_specs=[
            pl.BlockSpec((2, BR, 128), lambda i: (0, i, 0)),
            pl.BlockSpec((BR // 128, 128), lambda i: (i, 0)),
        ],
        out_shape=[
            jax.ShapeDtypeStruct((2, N_PROC, 128), jnp.float32),
            jax.ShapeDtypeStruct((N_PROC // 128, 128), jnp.int32),
        ],
    )(feats, ctf, invs, wpreT, gamma, beta, wpos4)


def _shift1(a, k):
    z = list(a.shape)
    z[-2] = abs(k)
    zer = jnp.zeros(z, a.dtype)
    if k > 0:
        return jnp.concatenate([a[..., k:, :], zer], axis=-2)
    return jnp.concatenate([zer, a[..., :k, :]], axis=-2)


def _box_zy(x):
    s = x + _shift1(x, 1) + _shift1(x, -1)
    return s + _shift1(s, G1) + _shift1(s, -G1)


def _k3_body(g_ref, c_ref, o_ref, scrf, scrc):
    s = _box_zy(g_ref[...])
    t = _box_zy(c_ref[...])
    s0 = s[:, :PLANE]
    s1 = s[:, PLANE:]
    t0 = t[:PLANE]
    t1 = t[PLANE:]
    outf0 = scrf[2] + scrf[0] + scrf[1]
    outf1 = scrf[0] + scrf[1] + s0
    outc0 = scrc[2] + scrc[0] + scrc[1]
    outc1 = scrc[0] + scrc[1] + t0
    a0 = outf0 / outc0[None, :, 0:1]
    a1 = outf1 / outc1[None, :, 0:1]
    o_ref[...] = jnp.concatenate([a0, a1], axis=1)
    scrf[2] = scrf[1]
    scrf[0] = s0
    scrf[1] = s1
    scrc[2] = scrc[1]
    scrc[0] = t0
    scrc[1] = t1


def _k3(grid3, gc):
    nblk = NPLANES // 2
    return pl.pallas_call(
        _k3_body,
        grid=(nblk + 1,),
        in_specs=[
            pl.BlockSpec((2, 2 * PLANE, 128),
                         lambda i: (0, jnp.minimum(i, nblk - 1), 0)),
            pl.BlockSpec((2 * PLANE, 8),
                         lambda i: (jnp.minimum(i, nblk - 1), 0)),
        ],
        out_specs=pl.BlockSpec((2, 2 * PLANE, 128),
                               lambda i: (0, jnp.maximum(i - 1, 0), 0)),
        out_shape=jax.ShapeDtypeStruct((2, V_PAD, 128), jnp.float32),
        scratch_shapes=[
            pltpu.VMEM((3, 2, PLANE, 128), jnp.float32),
            pltpu.VMEM((3, PLANE, 8), jnp.float32),
        ],
    )(grid3, gc)


def _k5_body(lo_ref, hi_ref, ct_ref, wpos_ref, o_ref):
    pos = _pos_of(ct_ref[...], wpos_ref)
    sw, cw = _sincos(pos)
    o_ref[...] = lo_ref[...] * cw + hi_ref[...] * sw


def _k5(lo, hi, ctf, wpos4, n):
    return pl.pallas_call(
        _k5_body,
        grid=(NBLK,),
        in_specs=[
            pl.BlockSpec((BR, 128), lambda i: (i, 0)),
            pl.BlockSpec((BR, 128), lambda i: (i, 0)),
            pl.BlockSpec((4, BR), lambda i: (0, i)),
            pl.BlockSpec((4, 128), lambda i: (0, 0)),
        ],
        out_specs=pl.BlockSpec((BR, 128), lambda i: (i, 0)),
        out_shape=jax.ShapeDtypeStruct((n, 128), jnp.float32),
    )(lo, hi, ctf, wpos4)


def _k2_body(fw_hbm, idx_hbm, zz_hbm, ones_hbm, grid_hbm, gc_hbm,
             spg, zbuf, ibuf, vbufa, vbufb, onesbuf, obuf, obufb,
             sema, semb, sems):
    c = lax.axis_index("c")
    s = lax.axis_index("s")
    pltpu.sync_copy(zz_hbm, zbuf)
    pltpu.sync_copy(ones_hbm, onesbuf)
    pltpu.sync_copy(idx_hbm.at[pl.ds(s * 49, 49)], ibuf)
    row0 = s * VT
    p_base = s * PT

    def chunk_iter(jj, carry):
        chunk = jj * 2 + c
        valid = chunk < NCHUNK
        is_cnt = chunk == NCHUNK - 1
        half = chunk // 16
        col0 = (chunk % 16) * 8

        def _src(r):
            return fw_hbm.at[half, pl.ds(p_base + r * PR, PR),
                             pl.ds(col0, 8)]

        def _start(r, buf, sem):
            pltpu.async_copy(_src(r), buf, sem)

        def _wait(r, buf, sem):
            pltpu.make_async_copy(_src(r), buf, sem).wait()

        def _scat(r, buf):
            for q in range(7):
                pltpu.async_copy(buf.at[pl.ds(q * 128, 128)],
                                 spg.at[ibuf.at[r * 7 + q]], sems, add=True)
            for q in range(7):
                pltpu.make_async_copy(buf.at[pl.ds(q * 128, 128)],
                                      spg.at[ibuf.at[r * 7 + q]],
                                      sems).wait()

        @pl.when(valid)
        def _zero():
            for r in range(8):
                pltpu.async_copy(zbuf, spg.at[pl.ds(row0 + r * SUBZ, SUBZ)],
                                 sems)
            for r in range(8):
                pltpu.make_async_copy(zbuf,
                                      spg.at[pl.ds(row0 + r * SUBZ, SUBZ)],
                                      sems).wait()

        plsc.subcore_barrier()

        @pl.when(valid & jnp.logical_not(is_cnt))
        def _scatter():
            _start(0, vbufa, sema)

            def pair(g, cc):
                r = g * 2
                _start(r + 1, vbufb, semb)
                _wait(r, vbufa, sema)
                _scat(r, vbufa)
                _start(r + 2, vbufa, sema)
                _wait(r + 1, vbufb, semb)
                _scat(r + 1, vbufb)
                return cc
            lax.fori_loop(0, SR // 2, pair, 0)
            _wait(SR - 1, vbufa, sema)
            _scat(SR - 1, vbufa)

        @pl.when(is_cnt)
        def _scatter_ones():
            def sc1(r, cc):
                for q in range(7):
                    pltpu.async_copy(onesbuf, spg.at[ibuf.at[r * 7 + q]],
                                     sems, add=True)
                for q in range(7):
                    pltpu.make_async_copy(onesbuf,
                                          spg.at[ibuf.at[r * 7 + q]],
                                          sems).wait()
                return cc
            lax.fori_loop(0, 7, sc1, 0)

        plsc.subcore_barrier()

        def _ro_start(r, buf, sem):
            pltpu.async_copy(spg.at[pl.ds(row0 + r * SUBZ, SUBZ)], buf, sem)

        def _ro_wait(r, buf, sem):
            pltpu.make_async_copy(spg.at[pl.ds(row0 + r * SUBZ, SUBZ)],
                                  buf, sem).wait()

        @pl.when(valid & jnp.logical_not(is_cnt))
        def _readout():
            def _fl(r, buf):
                pltpu.sync_copy(buf, grid_hbm.at[half,
                                                 pl.ds(row0 + r * SUBZ, SUBZ),
                                                 pl.ds(col0, 8)])
            _ro_start(0, obuf, sema)

            def rp(g, cc):
                r = g * 2
                _ro_start(r + 1, obufb, semb)
                _ro_wait(r, obuf, sema)
                _fl(r, obuf)
                _ro_start(r + 2, obuf, sema)
                _ro_wait(r + 1, obufb, semb)
                _fl(r + 1, obufb)
                return cc
            lax.fori_loop(0, 3, rp, 0)
            _ro_start(7, obufb, semb)
            _ro_wait(6, obuf, sema)
            _fl(6, obuf)
            _ro_wait(7, obufb, semb)
            _fl(7, obufb)

        @pl.when(is_cnt)
        def _readout_cnt():
            def ro(r, cc):
                _ro_wait(r, obuf, sema)
                pltpu.sync_copy(obuf, gc_hbm.at[pl.ds(row0 + r * SUBZ, SUBZ)])

                @pl.when(r < 7)
                def _():
                    _ro_start(r + 1, obuf, sema)
                return cc
            _ro_start(0, obuf, sema)
            lax.fori_loop(0, 8, ro, 0)

        plsc.subcore_barrier()
        return carry

    lax.fori_loop(0, 17, chunk_iter, 0)


@functools.cache
def _k2():
    return pl.kernel(
        _k2_body,
        out_type=[
            jax.ShapeDtypeStruct((2, V_PAD, 128), jnp.float32),
            jax.ShapeDtypeStruct((V_PAD, 8), jnp.float32),
        ],
        mesh=_mesh(),
        compiler_params=pltpu.CompilerParams(use_tc_tiling_on_sc=False),
        scratch_types=[
            pltpu.VMEM_SHARED((V_PAD, 8), jnp.float32),
            pltpu.VMEM((SUBZ, 8), jnp.float32),
            pltpu.VMEM((49, 128), jnp.int32),
            pltpu.VMEM((PR, 8), jnp.float32),
            pltpu.VMEM((PR, 8), jnp.float32),
            pltpu.VMEM((128, 8), jnp.float32),
            pltpu.VMEM((SUBZ, 8), jnp.float32),
            pltpu.VMEM((SUBZ, 8), jnp.float32),
            pltpu.SemaphoreType.DMA,
            pltpu.SemaphoreType.DMA,
            pltpu.SemaphoreType.DMA,
        ],
    )


def _k4_body(anorm_hbm, idx_hbm, lo_hbm, hi_hbm,
             ibuf, ibuf2, rbufa, rbufb, sema, semb):
    c = lax.axis_index("c")
    s = lax.axis_index("s")
    wid = s * 2 + c

    @pl.when(wid < NW4)
    def _work():
        base = wid * PW
        nch = PW // 128
        off = lax.rem(wid * nch, 8)
        abase = pl.multiple_of(wid * nch - off, 8)
        pltpu.sync_copy(idx_hbm.at[pl.ds(abase, 32)], ibuf)

        def addv(i, cc):
            for q in range(8):
                ibuf2[i, pl.ds(q * 16, 16)] = (
                    ibuf[i, pl.ds(q * 16, 16)] + V_PAD)
            return cc
        lax.fori_loop(0, 32, addv, 0)

        def _start(k, buf, sem):
            pltpu.async_copy(anorm_hbm.at[ibuf.at[off + k]], buf.at[0], sem)
            pltpu.async_copy(anorm_hbm.at[ibuf2.at[off + k]], buf.at[1], sem)

        def _wait(k, buf, sem):
            pltpu.make_async_copy(anorm_hbm.at[ibuf.at[off + k]],
                                  buf.at[0], sem).wait()
            pltpu.make_async_copy(anorm_hbm.at[ibuf2.at[off + k]],
                                  buf.at[1], sem).wait()

        def _flush(k, buf):
            pltpu.sync_copy(buf.at[0], lo_hbm.at[pl.ds(base + k * 128, 128)])
            pltpu.sync_copy(buf.at[1], hi_hbm.at[pl.ds(base + k * 128, 128)])

        _start(0, rbufa, sema)

        def pair(g, carry):
            j = g * 2
            _start(j + 1, rbufb, semb)
            _wait(j, rbufa, sema)
            _flush(j, rbufa)
            _start(j + 2, rbufa, sema)
            _wait(j + 1, rbufb, semb)
            _flush(j + 1, rbufb)
            return carry

        lax.fori_loop(0, nch // 2 - 1, pair, 0)
        _start(nch - 1, rbufb, semb)
        _wait(nch - 2, rbufa, sema)
        _flush(nch - 2, rbufa)
        _wait(nch - 1, rbufb, semb)
        _flush(nch - 1, rbufb)


@functools.cache
def _k4():
    return pl.kernel(
        _k4_body,
        out_type=[
            jax.ShapeDtypeStruct((N_PROC, 128), jnp.float32),
            jax.ShapeDtypeStruct((N_PROC, 128), jnp.float32),
        ],
        mesh=_mesh(),
        compiler_params=pltpu.CompilerParams(use_tc_tiling_on_sc=True),
        scratch_types=[
            pltpu.VMEM((32, 128), jnp.int32),
            pltpu.VMEM((32, 128), jnp.int32),
            pltpu.VMEM((2, 128, 128), jnp.float32),
            pltpu.VMEM((2, 128, 128), jnp.float32),
            pltpu.SemaphoreType.DMA,
            pltpu.SemaphoreType.DMA,
        ],
    )


def kernel(feats, coords, W_pre, ln_gamma, ln_beta, W_pos, stride):
    n = feats.shape[0]
    ctf = coords.T.astype(jnp.float32)
    padcol = jnp.tile(jnp.array([[0.0], [0.0], [0.0], [4.0]], jnp.float32),
                      (1, N_PROC - n))
    ctf = jnp.concatenate([ctf, padcol], axis=1)
    invs = jnp.reshape(1.0 / jnp.asarray(stride, jnp.float32), (1, 1))
    wpreT = W_pre.T
    wpos4 = jnp.pad(W_pos.T.astype(jnp.float32), ((0, 1), (0, 0)))
    gamma = ln_gamma.reshape(1, 128)
    beta = ln_beta.reshape(1, 128)
    zz = jnp.zeros((SUBZ, 8), jnp.float32)
    cones = jnp.zeros((128, 8), jnp.float32).at[:, 0].set(1.0)

    fw, idx2 = _k1(feats, ctf, invs, wpreT, gamma, beta, wpos4)
    grid3, gc = _k2()(fw, idx2, zz, cones)
    anorm3 = _k3(grid3, gc)
    anorm2 = anorm3.reshape(2 * V_PAD, 128)
    lo, hi = _k4()(anorm2, idx2)
    return _k5(lo, hi, ctf, wpos4, n)
